# baseline (speedup 1.0000x reference)
import functools

import jax
import jax.numpy as jnp
from jax import lax
from jax.experimental import pallas as pl
from jax.experimental.pallas import tpu as pltpu
from jax.experimental.pallas import tpu_sc as plsc

_B = 16384
_D = 64
_C = 1000
_L = 16
_NC = 2
_NS = 16
_NW = _NC * _NS
_BW = _B // _NW
_BH = _B // _NS
_SG = 64

_CENT_CHUNKS = [(i * _SG, min(_SG, _C - i * _SG))
                for i in range((_C + _SG - 1) // _SG)]
_FEAT_CHUNKS = [(i * _SG, min(_SG, _BW - i * _SG))
                for i in range((_BW + _SG - 1) // _SG)]


def _repack(stage_v, pk_v, base, nrows):
    @plsc.parallel_loop(0, nrows, unroll=8)
    def _(r):
        for k in range(_D // _L):
            pk_v[pl.ds((base + r) * _D + k * _L, _L)] = (
                stage_v[r, pl.ds(k * _L, _L)])


def _body(features_hbm, labels_hbm, centers_hbm, out_hbm,
          cent_pk_v, feat_pk_v, stage0_v, stage1_v,
          lab_hist_v, lab_my_v, hist_v, slab_v, inv_v,
          hist_stage_s, inv_s, sem0, sem1):
    cid = lax.axis_index("c")
    sid = lax.axis_index("s")
    wid = cid * _NS + sid

    stages = (stage0_v, stage1_v)
    sems = (sem0, sem1)

    chunks = ([(centers_hbm, b, n) for (b, n) in _CENT_CHUNKS]
              + [(features_hbm, wid * _BW + b, n) for (b, n) in _FEAT_CHUNKS])

    def start(i):
        src, b, n = chunks[i]
        return pltpu.async_copy(
            src.at[pl.ds(b, n)], stages[i % 2].at[pl.ds(0, n)], sems[i % 2])

    cp = {0: start(0)}

    with jax.named_scope("ph1_labels_dma"):
        pltpu.sync_copy(labels_hbm.at[pl.ds(sid * _BH, _BH)], lab_hist_v)
        pltpu.sync_copy(labels_hbm.at[pl.ds(wid * _BW, _BW)], lab_my_v)

    with jax.named_scope("ph1_hist"):
        zero = jnp.zeros((_L,), jnp.float32)
        for r in range(8):
            for c in range(8):
                hist_v[r, pl.ds(c * _L, _L)] = zero

        ones = jnp.ones((_L,), jnp.float32)

        def hist_step(i, _):
            idx = lab_hist_v[pl.ds(i * _L, _L)]
            plsc.addupdate_scatter(
                hist_v, [lax.shift_right_logical(idx, 7), idx & 127], ones)
            return 0
        lax.fori_loop(0, _BH // _L, hist_step, 0)

    with jax.named_scope("ph1_allreduce"):
        pltpu.sync_copy(hist_v, hist_stage_s.at[sid])
        plsc.subcore_barrier()

        @pl.when(sid < 8)
        def _():
            pltpu.sync_copy(hist_stage_s.at[:, sid], slab_v)
            for k in range(8):
                sl = pl.ds(k * _L, _L)
                def add_row(r, a):
                    return a + slab_v[r, sl]
                n = lax.fori_loop(1, _NS, add_row, slab_v[0, sl])
                inv_v[0, sl] = jnp.where(n > 0.0, 1.0 / n, 0.0)
            pltpu.sync_copy(inv_v.at[0], inv_s.at[sid])
        plsc.subcore_barrier()
        pltpu.sync_copy(inv_s, inv_v)

    with jax.named_scope("ph2_repack"):
        ncc = len(_CENT_CHUNKS)
        for i, (_, b, n) in enumerate(chunks):
            cp[i].wait()
            if i + 1 < len(chunks):
                cp[i + 1] = start(i + 1)
            if i < ncc:
                _repack(stages[i % 2], cent_pk_v, _CENT_CHUNKS[i][0], n)
            else:
                _repack(stages[i % 2], feat_pk_v,
                        _FEAT_CHUNKS[i - ncc][0], n)

    with jax.named_scope("ph3_main"):
        zero = jnp.zeros((_L,), jnp.float32)

        @plsc.parallel_loop(0, _BW // _L, carry=(zero, zero, zero, zero))
        def acc_loop(i, carry):
            accs = list(carry)
            idx = lab_my_v[pl.ds(i * _L, _L)]
            inv16 = plsc.load_gather(
                inv_v, [lax.shift_right_logical(idx, 7), idx & 127])
            for j in range(_L):
                l = idx[j]
                inv_j = inv16[j]
                fb = (i * _L + j) * _D
                cb = l * _D
                sq = []
                for k in range(_D // _L):
                    dlt = (feat_pk_v[pl.ds(fb + k * _L, _L)]
                           - cent_pk_v[pl.ds(cb + k * _L, _L)])
                    sq.append(dlt * dlt)
                s = (sq[0] + sq[1]) + (sq[2] + sq[3])
                accs[j % 4] = accs[j % 4] + s * inv_j
            return tuple(accs)
        a0, a1, a2, a3 = acc_loop
        acc = (a0 + a1) + (a2 + a3)

    hist_v[0, pl.ds(0, _L)] = acc
    pltpu.sync_copy(hist_v, hist_stage_s.at[sid])
    plsc.subcore_barrier()

    @pl.when(sid == 0)
    def _():
        pltpu.sync_copy(hist_stage_s.at[:, 0], slab_v)
        def add_part(r, a):
            return a + slab_v[r, pl.ds(0, _L)]
        tot = lax.fori_loop(1, _NS, add_part, slab_v[0, pl.ds(0, _L)])
        hist_v[0, pl.ds(0, _L)] = tot * (1.0 / (_D * _B))
        pltpu.sync_copy(hist_v, out_hbm.at[cid])


@jax.jit
def _center_loss_sc(features, labels, centers):
    mesh = plsc.VectorSubcoreMesh(core_axis_name="c", subcore_axis_name="s")
    out = pl.kernel(
        _body,
        out_type=jax.ShapeDtypeStruct((_NC, 8, 128), jnp.float32),
        mesh=mesh,
        compiler_params=pltpu.CompilerParams(
            needs_layout_passes=False, use_tc_tiling_on_sc=True),
        scratch_types=[
            pltpu.VMEM((_C * _D,), jnp.float32),
            pltpu.VMEM((_BW * _D,), jnp.float32),
            pltpu.VMEM((_SG, _D), jnp.float32),
            pltpu.VMEM((_SG, _D), jnp.float32),
            pltpu.VMEM((_BH,), jnp.int32),
            pltpu.VMEM((_BW,), jnp.int32),
            pltpu.VMEM((8, 128), jnp.float32),
            pltpu.VMEM((_NS, 128), jnp.float32),
            pltpu.VMEM((8, 128), jnp.float32),
            pltpu.VMEM_SHARED((_NS, 8, 128), jnp.float32),
            pltpu.VMEM_SHARED((8, 128), jnp.float32),
            pltpu.SemaphoreType.DMA,
            pltpu.SemaphoreType.DMA,
        ],
    )(features, labels, centers)
    return jnp.sum(out[:, 0, :_L])


def kernel(features, labels, centers):
    labels = labels.reshape(-1).astype(jnp.int32)
    return _center_loss_sc(features, labels, centers)


# speedup vs baseline: 1.5643x; 1.5643x over previous
import functools

import jax
import jax.numpy as jnp
from jax import lax
from jax.experimental import pallas as pl
from jax.experimental.pallas import tpu as pltpu
from jax.experimental.pallas import tpu_sc as plsc

_B = 16384
_D = 64
_C = 1000
_L = 16
_NC = 2
_NS = 16
_NW = _NC * _NS
_BW = _B // _NW
_BH = _B // _NS
_FC = 128
_NFC = _BW // _FC
_CS = 64


def _body(features_hbm, labels_hbm, centers_hbm, out_hbm,
          cent_pk_v, pk_local_v, stage0_v, stage1_v,
          lab_hist_v, lab_my_v, hist_v, slab_v, inv_v,
          hist_stage_s, inv_s, cent_pk_s, sem0, sem1, sem2):
    cid = lax.axis_index("c")
    sid = lax.axis_index("s")
    wid = cid * _NS + sid

    stages = (stage0_v, stage1_v)
    sems = (sem0, sem1)

    cbase = jnp.minimum(sid * _CS, _C - _CS)
    cp_slab = pltpu.async_copy(
        centers_hbm.at[pl.ds(cbase, _CS)], stage0_v.at[pl.ds(0, _CS)], sem0)
    cp_feat = pltpu.async_copy(
        features_hbm.at[pl.ds(wid * _BW, _FC)], stage1_v, sem1)

    with jax.named_scope("ph1_labels_dma"):
        pltpu.sync_copy(labels_hbm.at[pl.ds(sid * _BH, _BH)], lab_hist_v)
        pltpu.sync_copy(labels_hbm.at[pl.ds(wid * _BW, _BW)], lab_my_v)

    with jax.named_scope("ph1_hist"):
        zero = jnp.zeros((_L,), jnp.float32)
        for r in range(8):
            for c in range(8):
                hist_v[r, pl.ds(c * _L, _L)] = zero

        ones = jnp.ones((_L,), jnp.float32)

        def hist_step(i, _):
            idx = lab_hist_v[pl.ds(i * _L, _L)]
            plsc.addupdate_scatter(
                hist_v, [lax.shift_right_logical(idx, 7), idx & 127], ones)
            return 0
        lax.fori_loop(0, _BH // _L, hist_step, 0)

    with jax.named_scope("ph1_allreduce"):
        pltpu.sync_copy(hist_v, hist_stage_s.at[sid])
        plsc.subcore_barrier()

        @pl.when(sid < 8)
        def _():
            pltpu.sync_copy(hist_stage_s.at[:, sid], slab_v)
            for k in range(8):
                sl = pl.ds(k * _L, _L)
                def add_row(r, a):
                    return a + slab_v[r, sl]
                n = lax.fori_loop(1, _NS, add_row, slab_v[0, sl])
                inv_v[0, sl] = jnp.where(n > 0.0, 1.0 / n, 0.0)
            pltpu.sync_copy(inv_v.at[0], inv_s.at[sid])

    with jax.named_scope("ph2_repack"):
        cp_slab.wait()

        @plsc.parallel_loop(0, _CS, unroll=4)
        def _(r):
            for k in range(_D // _L):
                pk_local_v[pl.ds(r * _D + k * _L, _L)] = (
                    stage0_v[r, pl.ds(k * _L, _L)])

        pltpu.sync_copy(pk_local_v, cent_pk_s.at[pl.ds(cbase * _D, _CS * _D)])
        plsc.subcore_barrier()
        cp_cent = pltpu.async_copy(cent_pk_s, cent_pk_v, sem2)
        pltpu.sync_copy(inv_s, inv_v)
        cp_cent.wait()

    with jax.named_scope("ph3_main"):
        zero = jnp.zeros((_L,), jnp.float32)
        accs = (zero, zero, zero, zero)
        for c in range(_NFC):
            cp_feat.wait()
            buf = stages[(c + 1) % 2]
            if c + 1 < _NFC:
                cp_feat = pltpu.async_copy(
                    features_hbm.at[pl.ds(wid * _BW + (c + 1) * _FC, _FC)],
                    stages[c % 2], sems[c % 2])

            @plsc.parallel_loop(0, _FC // _L, carry=accs)
            def acc_loop(i, carry, c=c, buf=buf):
                a = list(carry)
                idx = lab_my_v[pl.ds(c * _FC + i * _L, _L)]
                inv16 = plsc.load_gather(
                    inv_v, [lax.shift_right_logical(idx, 7), idx & 127])
                for j in range(_L):
                    l = idx[j]
                    inv_j = inv16[j]
                    cb = l * _D
                    sq = []
                    for k in range(_D // _L):
                        dlt = (buf[i * _L + j, pl.ds(k * _L, _L)]
                               - cent_pk_v[pl.ds(cb + k * _L, _L)])
                        sq.append(dlt * dlt)
                    s = (sq[0] + sq[1]) + (sq[2] + sq[3])
                    a[j % 4] = a[j % 4] + s * inv_j
                return tuple(a)
            accs = acc_loop
        a0, a1, a2, a3 = accs
        acc = (a0 + a1) + (a2 + a3)

    hist_v[0, pl.ds(0, _L)] = acc
    pltpu.sync_copy(hist_v, hist_stage_s.at[sid])
    plsc.subcore_barrier()

    @pl.when(sid == 0)
    def _():
        pltpu.sync_copy(hist_stage_s.at[:, 0], slab_v)
        def add_part(r, a):
            return a + slab_v[r, pl.ds(0, _L)]
        tot = lax.fori_loop(1, _NS, add_part, slab_v[0, pl.ds(0, _L)])
        hist_v[0, pl.ds(0, _L)] = tot * (1.0 / (_D * _B))
        pltpu.sync_copy(hist_v, out_hbm.at[cid])


@jax.jit
def _center_loss_sc(features, labels, centers):
    mesh = plsc.VectorSubcoreMesh(core_axis_name="c", subcore_axis_name="s")
    out = pl.kernel(
        _body,
        out_type=jax.ShapeDtypeStruct((_NC, 8, 128), jnp.float32),
        mesh=mesh,
        compiler_params=pltpu.CompilerParams(
            needs_layout_passes=False, use_tc_tiling_on_sc=True),
        scratch_types=[
            pltpu.VMEM((_C * _D,), jnp.float32),
            pltpu.VMEM((_CS * _D,), jnp.float32),
            pltpu.VMEM((_FC, _D), jnp.float32),
            pltpu.VMEM((_FC, _D), jnp.float32),
            pltpu.VMEM((_BH,), jnp.int32),
            pltpu.VMEM((_BW,), jnp.int32),
            pltpu.VMEM((8, 128), jnp.float32),
            pltpu.VMEM((_NS, 128), jnp.float32),
            pltpu.VMEM((8, 128), jnp.float32),
            pltpu.VMEM_SHARED((_NS, 8, 128), jnp.float32),
            pltpu.VMEM_SHARED((8, 128), jnp.float32),
            pltpu.VMEM_SHARED((_C * _D,), jnp.float32),
            pltpu.SemaphoreType.DMA,
            pltpu.SemaphoreType.DMA,
            pltpu.SemaphoreType.DMA,
        ],
    )(features, labels, centers)
    return jnp.sum(out[:, 0, :_L])


def kernel(features, labels, centers):
    labels = labels.reshape(-1).astype(jnp.int32)
    return _center_loss_sc(features, labels, centers)


# speedup vs baseline: 1.8599x; 1.1889x over previous
import functools

import jax
import jax.numpy as jnp
from jax import lax
from jax.experimental import pallas as pl
from jax.experimental.pallas import tpu as pltpu
from jax.experimental.pallas import tpu_sc as plsc

_B = 16384
_D = 64
_C = 1000
_CP = 1024
_L = 16
_NC = 2
_NS = 16
_NW = _NC * _NS
_BW = _B // _NW
_BH = _B // _NS


def _body(featt_hbm, labels_hbm, centt_hbm, out_hbm,
          cent_pk_v, pk_local_v, stage_v, feat_v,
          lab_hist_v, lab_my_v, hist_v, slab_v, inv_v,
          hist_stage_s, inv_s, cent_pk_s, sem0, sem1):
    cid = lax.axis_index("c")
    sid = lax.axis_index("s")
    wid = cid * _NS + sid

    cp_feat = pltpu.async_copy(
        featt_hbm.at[:, pl.ds(wid * _BW, _BW)], feat_v, sem0)
    cp_stripe = pltpu.async_copy(
        centt_hbm.at[pl.ds(jnp.minimum(sid, 7) * 8, 8)], stage_v, sem1)

    with jax.named_scope("ph1_labels_dma"):
        pltpu.sync_copy(labels_hbm.at[pl.ds(sid * _BH, _BH)], lab_hist_v)
        pltpu.sync_copy(labels_hbm.at[pl.ds(wid * _BW, _BW)], lab_my_v)

    with jax.named_scope("ph1_hist"):
        zero = jnp.zeros((_L,), jnp.float32)
        for r in range(8):
            for c in range(8):
                hist_v[r, pl.ds(c * _L, _L)] = zero

        ones = jnp.ones((_L,), jnp.float32)

        def hist_step(i, _):
            idx = lab_hist_v[pl.ds(i * _L, _L)]
            plsc.addupdate_scatter(
                hist_v, [lax.shift_right_logical(idx, 7), idx & 127], ones)
            return 0
        lax.fori_loop(0, _BH // _L, hist_step, 0)

    with jax.named_scope("ph1_allreduce"):
        pltpu.sync_copy(hist_v, hist_stage_s.at[sid])
        plsc.subcore_barrier()

        @pl.when(sid < 8)
        def _():
            pltpu.sync_copy(hist_stage_s.at[:, sid], slab_v)
            for k in range(8):
                sl = pl.ds(k * _L, _L)
                def add_row(r, a):
                    return a + slab_v[r, sl]
                n = lax.fori_loop(1, _NS, add_row, slab_v[0, sl])
                inv_v[0, sl] = jnp.where(n > 0.0, 1.0 / n, 0.0)
            pltpu.sync_copy(inv_v.at[0], inv_s.at[sid])

    with jax.named_scope("ph2_repack"):
        cp_stripe.wait()

        @pl.when(sid < 8)
        def _():
            offs = list(range(0, _C - _L + 1, _L)) + [_C - _L]
            for r in range(8):
                for off in offs:
                    pk_local_v[pl.ds(r * _CP + off, _L)] = (
                        stage_v[r, pl.ds(off, _L)])
            pltpu.sync_copy(
                pk_local_v,
                cent_pk_s.at[pl.ds(jnp.minimum(sid, 7) * 8 * _CP, 8 * _CP)])
        plsc.subcore_barrier()
        cp_cent = pltpu.async_copy(cent_pk_s, cent_pk_v, sem1)
        pltpu.sync_copy(inv_s, inv_v)
        cp_cent.wait()
        cp_feat.wait()

    with jax.named_scope("ph3_main"):
        zero = jnp.zeros((_L,), jnp.float32)

        @plsc.parallel_loop(0, _BW // _L, carry=(zero, zero, zero, zero))
        def acc_loop(i, carry):
            a = list(carry)
            idx = lab_my_v[pl.ds(i * _L, _L)]
            inv16 = plsc.load_gather(
                inv_v, [lax.shift_right_logical(idx, 7), idx & 127])
            ps = [zero, zero, zero, zero]
            for k in range(_D):
                f = feat_v[k, pl.ds(i * _L, _L)]
                cv = plsc.load_gather(cent_pk_v, [idx + (k * _CP)])
                dlt = f - cv
                ps[k % 4] = ps[k % 4] + dlt * dlt
            return (a[0] + (ps[0] + ps[1]) * inv16,
                    a[1] + (ps[2] + ps[3]) * inv16,
                    a[2], a[3])
        a0, a1, a2, a3 = acc_loop
        acc = (a0 + a1) + (a2 + a3)

    hist_v[0, pl.ds(0, _L)] = acc
    pltpu.sync_copy(hist_v, hist_stage_s.at[sid])
    plsc.subcore_barrier()

    @pl.when(sid == 0)
    def _():
        pltpu.sync_copy(hist_stage_s.at[:, 0], slab_v)
        def add_part(r, a):
            return a + slab_v[r, pl.ds(0, _L)]
        tot = lax.fori_loop(1, _NS, add_part, slab_v[0, pl.ds(0, _L)])
        hist_v[0, pl.ds(0, _L)] = tot * (1.0 / (_D * _B))
        pltpu.sync_copy(hist_v, out_hbm.at[cid])


@jax.jit
def _center_loss_sc(features, labels, centers):
    mesh = plsc.VectorSubcoreMesh(core_axis_name="c", subcore_axis_name="s")
    out = pl.kernel(
        _body,
        out_type=jax.ShapeDtypeStruct((_NC, 8, 128), jnp.float32),
        mesh=mesh,
        compiler_params=pltpu.CompilerParams(
            needs_layout_passes=False, use_tc_tiling_on_sc=True),
        scratch_types=[
            pltpu.VMEM((_D * _CP,), jnp.float32),
            pltpu.VMEM((8 * _CP,), jnp.float32),
            pltpu.VMEM((8, _C), jnp.float32),
            pltpu.VMEM((_D, _BW), jnp.float32),
            pltpu.VMEM((_BH,), jnp.int32),
            pltpu.VMEM((_BW,), jnp.int32),
            pltpu.VMEM((8, 128), jnp.float32),
            pltpu.VMEM((_NS, 128), jnp.float32),
            pltpu.VMEM((8, 128), jnp.float32),
            pltpu.VMEM_SHARED((_NS, 8, 128), jnp.float32),
            pltpu.VMEM_SHARED((8, 128), jnp.float32),
            pltpu.VMEM_SHARED((_D * _CP,), jnp.float32),
            pltpu.SemaphoreType.DMA,
            pltpu.SemaphoreType.DMA,
        ],
    )(features, labels, centers)
    return jnp.sum(out[:, 0, :_L])


def kernel(features, labels, centers):
    labels = labels.reshape(-1).astype(jnp.int32)
    return _center_loss_sc(features.T, labels, centers.T)


# speedup vs baseline: 1.9024x; 1.0229x over previous
import functools

import jax
import jax.numpy as jnp
from jax import lax
from jax.experimental import pallas as pl
from jax.experimental.pallas import tpu as pltpu
from jax.experimental.pallas import tpu_sc as plsc

_B = 16384
_D = 64
_C = 1000
_CP = 1024
_L = 16
_NC = 2
_NS = 16
_NW = _NC * _NS
_BW = _B // _NW
_BH = _B // _NS


def _body(featt_hbm, labels_hbm, centt_hbm, out_hbm,
          cent_pk_v, pk_local_v, stage_v, feat_v,
          lab_hist_v, lab_my_v, hist_v, slab_v, inv_v,
          hist_stage_s, inv_s, cent_pk_s, sem0, sem1, sem2):
    cid = lax.axis_index("c")
    sid = lax.axis_index("s")
    wid = cid * _NS + sid

    cp_feat = pltpu.async_copy(
        featt_hbm.at[:, pl.ds(wid * _BW, _BW)], feat_v, sem0)
    cp_stripe = pltpu.async_copy(
        centt_hbm.at[pl.ds(jnp.minimum(sid, 7) * 8, 8)], stage_v, sem1)

    with jax.named_scope("ph1_labels_dma"):
        cp_lab = pltpu.async_copy(
            labels_hbm.at[pl.ds(wid * _BW, _BW)], lab_my_v, sem2)
        pltpu.sync_copy(labels_hbm.at[pl.ds(sid * _BH, _BH)], lab_hist_v)
        cp_lab.wait()

    with jax.named_scope("ph1_hist"):
        zero = jnp.zeros((_L,), jnp.float32)
        for r in range(8):
            for c in range(8):
                hist_v[r, pl.ds(c * _L, _L)] = zero

        ones = jnp.ones((_L,), jnp.float32)

        def hist_step(i, _):
            idx = lab_hist_v[pl.ds(i * _L, _L)]
            plsc.addupdate_scatter(
                hist_v, [lax.shift_right_logical(idx, 7), idx & 127], ones)
            return 0
        lax.fori_loop(0, _BH // _L, hist_step, 0)

    with jax.named_scope("ph1_allreduce"):
        pltpu.sync_copy(hist_v, hist_stage_s.at[sid])
        plsc.subcore_barrier()

        @pl.when(sid < 8)
        def _():
            pltpu.sync_copy(hist_stage_s.at[:, sid], slab_v)
            for k in range(8):
                sl = pl.ds(k * _L, _L)
                def add_row(r, a):
                    return a + slab_v[r, sl]
                n = lax.fori_loop(1, _NS, add_row, slab_v[0, sl])
                inv_v[0, sl] = jnp.where(n > 0.0, 1.0 / n, 0.0)
            pltpu.sync_copy(inv_v.at[0], inv_s.at[sid])

    with jax.named_scope("ph2_repack"):
        cp_stripe.wait()

        @pl.when(sid < 8)
        def _():
            offs = list(range(0, _C - _L + 1, _L)) + [_C - _L]
            for r in range(8):
                for off in offs:
                    pk_local_v[pl.ds(r * _CP + off, _L)] = (
                        stage_v[r, pl.ds(off, _L)])
            pltpu.sync_copy(
                pk_local_v,
                cent_pk_s.at[pl.ds(jnp.minimum(sid, 7) * 8 * _CP, 8 * _CP)])
        plsc.subcore_barrier()
        cp_cent = pltpu.async_copy(cent_pk_s, cent_pk_v, sem1)
        pltpu.sync_copy(inv_s, inv_v)
        cp_cent.wait()
        cp_feat.wait()

    with jax.named_scope("ph3_main"):
        zero = jnp.zeros((_L,), jnp.float32)

        @plsc.parallel_loop(0, _BW // _L, carry=(zero, zero, zero, zero))
        def acc_loop(i, carry):
            a = list(carry)
            idx = lab_my_v[pl.ds(i * _L, _L)]
            inv16 = plsc.load_gather(
                inv_v, [lax.shift_right_logical(idx, 7), idx & 127])
            ps = [zero, zero, zero, zero]
            for k in range(_D):
                f = feat_v[k, pl.ds(i * _L, _L)]
                cv = plsc.load_gather(cent_pk_v, [idx + (k * _CP)])
                dlt = f - cv
                ps[k % 4] = ps[k % 4] + dlt * dlt
            return (a[0] + (ps[0] + ps[1]) * inv16,
                    a[1] + (ps[2] + ps[3]) * inv16,
                    a[2], a[3])
        a0, a1, a2, a3 = acc_loop
        acc = (a0 + a1) + (a2 + a3)

    hist_v[0, pl.ds(0, _L)] = acc
    pltpu.sync_copy(hist_v, hist_stage_s.at[sid])
    plsc.subcore_barrier()

    @pl.when(sid == 0)
    def _():
        pltpu.sync_copy(hist_stage_s.at[:, 0], slab_v)
        def add_part(r, a):
            return a + slab_v[r, pl.ds(0, _L)]
        tot = lax.fori_loop(1, _NS, add_part, slab_v[0, pl.ds(0, _L)])
        hist_v[0, pl.ds(0, _L)] = tot * (1.0 / (_D * _B))
        pltpu.sync_copy(hist_v, out_hbm.at[cid])


@jax.jit
def _center_loss_sc(features, labels, centers):
    mesh = plsc.VectorSubcoreMesh(core_axis_name="c", subcore_axis_name="s")
    out = pl.kernel(
        _body,
        out_type=jax.ShapeDtypeStruct((_NC, 8, 128), jnp.float32),
        mesh=mesh,
        compiler_params=pltpu.CompilerParams(
            needs_layout_passes=False, use_tc_tiling_on_sc=True,
            skip_device_barrier=True),
        scratch_types=[
            pltpu.VMEM((_D * _CP,), jnp.float32),
            pltpu.VMEM((8 * _CP,), jnp.float32),
            pltpu.VMEM((8, _C), jnp.float32),
            pltpu.VMEM((_D, _BW), jnp.float32),
            pltpu.VMEM((_BH,), jnp.int32),
            pltpu.VMEM((_BW,), jnp.int32),
            pltpu.VMEM((8, 128), jnp.float32),
            pltpu.VMEM((_NS, 128), jnp.float32),
            pltpu.VMEM((8, 128), jnp.float32),
            pltpu.VMEM_SHARED((_NS, 8, 128), jnp.float32),
            pltpu.VMEM_SHARED((8, 128), jnp.float32),
            pltpu.VMEM_SHARED((_D * _CP,), jnp.float32),
            pltpu.SemaphoreType.DMA,
            pltpu.SemaphoreType.DMA,
            pltpu.SemaphoreType.DMA,
        ],
    )(features, labels, centers)
    return jnp.sum(out[:, 0, :_L])


def kernel(features, labels, centers):
    labels = labels.reshape(-1).astype(jnp.int32)
    return _center_loss_sc(features.T, labels, centers.T)


# speedup vs baseline: 1.9123x; 1.0052x over previous
import functools

import jax
import jax.numpy as jnp
from jax import lax
from jax.experimental import pallas as pl
from jax.experimental.pallas import tpu as pltpu
from jax.experimental.pallas import tpu_sc as plsc

_B = 16384
_D = 64
_C = 1000
_CP = 1024
_L = 16
_NC = 2
_NS = 16
_NW = _NC * _NS
_BW = _B // _NW
_BH = _B // _NS


def _body(featt_hbm, labels_hbm, centt_hbm, out_hbm,
          cent_pk_v, pk_local_v, stage_v, feat_v,
          lab_hist_v, lab_my_v, hist_v, slab_v, inv_v,
          hist_stage_s, inv_s, cent_pk_s, sem0, sem1, sem2):
    cid = lax.axis_index("c")
    sid = lax.axis_index("s")
    wid = cid * _NS + sid

    cp_feat = pltpu.async_copy(
        featt_hbm.at[:, pl.ds(wid * _BW, _BW)], feat_v, sem0)
    cp_stripe = pltpu.async_copy(
        centt_hbm.at[pl.ds(jnp.minimum(sid, 7) * 8, 8)], stage_v, sem1)

    with jax.named_scope("ph1_repack"):
        cp_lab = pltpu.async_copy(
            labels_hbm.at[pl.ds(wid * _BW, _BW)], lab_my_v, sem2)
        cp_stripe.wait()

        @pl.when(sid < 8)
        def _():
            offs = list(range(0, _C - _L + 1, _L)) + [_C - _L]
            for r in range(8):
                for off in offs:
                    pk_local_v[pl.ds(r * _CP + off, _L)] = (
                        stage_v[r, pl.ds(off, _L)])
            pltpu.sync_copy(
                pk_local_v,
                cent_pk_s.at[pl.ds(jnp.minimum(sid, 7) * 8 * _CP, 8 * _CP)])
        plsc.subcore_barrier()
        cp_cent = pltpu.async_copy(cent_pk_s, cent_pk_v, sem1)

    with jax.named_scope("ph1_hist"):
        pltpu.sync_copy(labels_hbm.at[pl.ds(sid * _BH, _BH)], lab_hist_v)
        zero = jnp.zeros((_L,), jnp.float32)
        for r in range(8):
            for c in range(8):
                hist_v[r, pl.ds(c * _L, _L)] = zero

        ones = jnp.ones((_L,), jnp.float32)

        def hist_step(i, _):
            idx = lab_hist_v[pl.ds(i * _L, _L)]
            plsc.addupdate_scatter(
                hist_v, [lax.shift_right_logical(idx, 7), idx & 127], ones)
            return 0
        lax.fori_loop(0, _BH // _L, hist_step, 0)

    with jax.named_scope("ph1_allreduce"):
        pltpu.sync_copy(hist_v, hist_stage_s.at[sid])
        plsc.subcore_barrier()

        @pl.when(sid < 8)
        def _():
            pltpu.sync_copy(hist_stage_s.at[:, sid], slab_v)
            for k in range(8):
                sl = pl.ds(k * _L, _L)
                def add_row(r, a):
                    return a + slab_v[r, sl]
                n = lax.fori_loop(1, _NS, add_row, slab_v[0, sl])
                inv_v[0, sl] = jnp.where(n > 0.0, 1.0 / n, 0.0)
            pltpu.sync_copy(inv_v.at[0], inv_s.at[sid])

    with jax.named_scope("ph2_wait"):
        plsc.subcore_barrier()
        pltpu.sync_copy(inv_s, inv_v)
        cp_lab.wait()
        cp_cent.wait()
        cp_feat.wait()

    with jax.named_scope("ph3_main"):
        zero = jnp.zeros((_L,), jnp.float32)

        @plsc.parallel_loop(0, _BW // _L, carry=(zero, zero, zero, zero))
        def acc_loop(i, carry):
            a = list(carry)
            idx = lab_my_v[pl.ds(i * _L, _L)]
            inv16 = plsc.load_gather(
                inv_v, [lax.shift_right_logical(idx, 7), idx & 127])
            ps = [zero, zero, zero, zero]
            for k in range(_D):
                f = feat_v[k, pl.ds(i * _L, _L)]
                cv = plsc.load_gather(cent_pk_v, [idx + (k * _CP)])
                dlt = f - cv
                ps[k % 4] = ps[k % 4] + dlt * dlt
            return (a[0] + (ps[0] + ps[1]) * inv16,
                    a[1] + (ps[2] + ps[3]) * inv16,
                    a[2], a[3])
        a0, a1, a2, a3 = acc_loop
        acc = (a0 + a1) + (a2 + a3)

    hist_v[0, pl.ds(0, _L)] = acc
    pltpu.sync_copy(hist_v, hist_stage_s.at[sid])
    plsc.subcore_barrier()

    @pl.when(sid == 0)
    def _():
        pltpu.sync_copy(hist_stage_s.at[:, 0], slab_v)
        def add_part(r, a):
            return a + slab_v[r, pl.ds(0, _L)]
        tot = lax.fori_loop(1, _NS, add_part, slab_v[0, pl.ds(0, _L)])
        hist_v[0, pl.ds(0, _L)] = tot * (1.0 / (_D * _B))
        pltpu.sync_copy(hist_v, out_hbm.at[cid])


@jax.jit
def _center_loss_sc(features, labels, centers):
    mesh = plsc.VectorSubcoreMesh(core_axis_name="c", subcore_axis_name="s")
    out = pl.kernel(
        _body,
        out_type=jax.ShapeDtypeStruct((_NC, 8, 128), jnp.float32),
        mesh=mesh,
        compiler_params=pltpu.CompilerParams(
            needs_layout_passes=False, use_tc_tiling_on_sc=True,
            skip_device_barrier=True),
        scratch_types=[
            pltpu.VMEM((_D * _CP,), jnp.float32),
            pltpu.VMEM((8 * _CP,), jnp.float32),
            pltpu.VMEM((8, _C), jnp.float32),
            pltpu.VMEM((_D, _BW), jnp.float32),
            pltpu.VMEM((_BH,), jnp.int32),
            pltpu.VMEM((_BW,), jnp.int32),
            pltpu.VMEM((8, 128), jnp.float32),
            pltpu.VMEM((_NS, 128), jnp.float32),
            pltpu.VMEM((8, 128), jnp.float32),
            pltpu.VMEM_SHARED((_NS, 8, 128), jnp.float32),
            pltpu.VMEM_SHARED((8, 128), jnp.float32),
            pltpu.VMEM_SHARED((_D * _CP,), jnp.float32),
            pltpu.SemaphoreType.DMA,
            pltpu.SemaphoreType.DMA,
            pltpu.SemaphoreType.DMA,
        ],
    )(features, labels, centers)
    return jnp.sum(out[:, 0, :_L])


def kernel(features, labels, centers):
    labels = labels.reshape(-1).astype(jnp.int32)
    return _center_loss_sc(features.T, labels, centers.T)


# speedup vs baseline: 1.9417x; 1.0154x over previous
import functools

import jax
import jax.numpy as jnp
from jax import lax
from jax.experimental import pallas as pl
from jax.experimental.pallas import tpu as pltpu
from jax.experimental.pallas import tpu_sc as plsc

_B = 16384
_D = 64
_C = 1000
_CP = 1024
_L = 16
_NC = 2
_NS = 16
_NW = _NC * _NS
_BW = _B // _NW
_BH = _B // _NS


def _body(featt_hbm, labels_hbm, centt_hbm, out_hbm,
          cent_pk_v, pk_local_v, stage_v, feat_v,
          lab_hist_v, lab_my_v, hist_v, slab_v, inv_v,
          hist_stage_s, inv_s, cent_pk_s, sem0, sem1, sem2):
    cid = lax.axis_index("c")
    sid = lax.axis_index("s")
    wid = cid * _NS + sid

    cp_feat = pltpu.async_copy(
        featt_hbm.at[:, pl.ds(wid * _BW, _BW)], feat_v, sem0)

    with jax.named_scope("ph1_repack"):
        cp_lab = pltpu.async_copy(
            labels_hbm.at[pl.ds(wid * _BW, _BW)], lab_my_v, sem2)

        @pl.when(sid < 8)
        def _():
            sbase = jnp.minimum(sid, 7) * 8
            pltpu.sync_copy(centt_hbm.at[pl.ds(sbase, 8)], stage_v)
            offs = list(range(0, _C - _L + 1, _L)) + [_C - _L]

            @plsc.parallel_loop(0, 8)
            def _(r):
                for off in offs:
                    pk_local_v[pl.ds(r * _CP + off, _L)] = (
                        stage_v[r, pl.ds(off, _L)])
            pltpu.sync_copy(pk_local_v, cent_pk_s.at[pl.ds(sbase * _CP,
                                                           8 * _CP)])
        plsc.subcore_barrier()
        cp_cent = pltpu.async_copy(cent_pk_s, cent_pk_v, sem1)

    with jax.named_scope("ph1_hist"):
        pltpu.sync_copy(labels_hbm.at[pl.ds(sid * _BH, _BH)], lab_hist_v)
        zero = jnp.zeros((_L,), jnp.float32)
        for r in range(8):
            for c in range(8):
                hist_v[r, pl.ds(c * _L, _L)] = zero

        ones = jnp.ones((_L,), jnp.float32)

        def hist_step(i, _):
            idx = lab_hist_v[pl.ds(i * _L, _L)]
            plsc.addupdate_scatter(
                hist_v, [lax.shift_right_logical(idx, 7), idx & 127], ones)
            return 0
        lax.fori_loop(0, _BH // _L, hist_step, 0)

    with jax.named_scope("ph1_allreduce"):
        pltpu.sync_copy(hist_v, hist_stage_s.at[sid])
        plsc.subcore_barrier()

        @pl.when(sid < 8)
        def _():
            pltpu.sync_copy(hist_stage_s.at[:, sid], slab_v)
            for k in range(8):
                sl = pl.ds(k * _L, _L)
                def add_row(r, a):
                    return a + slab_v[r, sl]
                n = lax.fori_loop(1, _NS, add_row, slab_v[0, sl])
                inv_v[0, sl] = jnp.where(n > 0.0, 1.0 / n, 0.0)
            pltpu.sync_copy(inv_v.at[0], inv_s.at[sid])

    with jax.named_scope("ph2_wait"):
        plsc.subcore_barrier()
        pltpu.sync_copy(inv_s, inv_v)
        cp_lab.wait()
        cp_cent.wait()
        cp_feat.wait()

    with jax.named_scope("ph3_main"):
        zero = jnp.zeros((_L,), jnp.float32)

        @plsc.parallel_loop(0, _BW // _L, carry=(zero, zero, zero, zero))
        def acc_loop(i, carry):
            a = list(carry)
            idx = lab_my_v[pl.ds(i * _L, _L)]
            inv16 = plsc.load_gather(
                inv_v, [lax.shift_right_logical(idx, 7), idx & 127])
            ps = [zero, zero, zero, zero]
            for k in range(_D):
                f = feat_v[k, pl.ds(i * _L, _L)]
                cv = plsc.load_gather(cent_pk_v, [idx + (k * _CP)])
                dlt = f - cv
                ps[k % 4] = ps[k % 4] + dlt * dlt
            return (a[0] + (ps[0] + ps[1]) * inv16,
                    a[1] + (ps[2] + ps[3]) * inv16,
                    a[2], a[3])
        a0, a1, a2, a3 = acc_loop
        acc = (a0 + a1) + (a2 + a3)

    hist_v[0, pl.ds(0, _L)] = acc
    pltpu.sync_copy(hist_v, hist_stage_s.at[sid])
    plsc.subcore_barrier()

    @pl.when(sid == 0)
    def _():
        pltpu.sync_copy(hist_stage_s.at[:, 0], slab_v)
        def add_part(r, a):
            return a + slab_v[r, pl.ds(0, _L)]
        tot = lax.fori_loop(1, _NS, add_part, slab_v[0, pl.ds(0, _L)])
        hist_v[0, pl.ds(0, _L)] = tot * (1.0 / (_D * _B))
        pltpu.sync_copy(hist_v, out_hbm.at[cid])


@jax.jit
def _center_loss_sc(features, labels, centers):
    mesh = plsc.VectorSubcoreMesh(core_axis_name="c", subcore_axis_name="s")
    out = pl.kernel(
        _body,
        out_type=jax.ShapeDtypeStruct((_NC, 8, 128), jnp.float32),
        mesh=mesh,
        compiler_params=pltpu.CompilerParams(
            needs_layout_passes=False, use_tc_tiling_on_sc=True,
            skip_device_barrier=True),
        scratch_types=[
            pltpu.VMEM((_D * _CP,), jnp.float32),
            pltpu.VMEM((8 * _CP,), jnp.float32),
            pltpu.VMEM((8, _C), jnp.float32),
            pltpu.VMEM((_D, _BW), jnp.float32),
            pltpu.VMEM((_BH,), jnp.int32),
            pltpu.VMEM((_BW,), jnp.int32),
            pltpu.VMEM((8, 128), jnp.float32),
            pltpu.VMEM((_NS, 128), jnp.float32),
            pltpu.VMEM((8, 128), jnp.float32),
            pltpu.VMEM_SHARED((_NS, 8, 128), jnp.float32),
            pltpu.VMEM_SHARED((8, 128), jnp.float32),
            pltpu.VMEM_SHARED((_D * _CP,), jnp.float32),
            pltpu.SemaphoreType.DMA,
            pltpu.SemaphoreType.DMA,
            pltpu.SemaphoreType.DMA,
        ],
    )(features, labels, centers)
    return jnp.sum(out[:, 0, :_L])


def kernel(features, labels, centers):
    labels = labels.reshape(-1).astype(jnp.int32)
    return _center_loss_sc(features.T, labels, centers.T)
